# baseline (speedup 1.0000x reference)
import functools
import math

import jax
import jax.numpy as jnp
from jax import lax
from jax.experimental import pallas as pl
from jax.experimental.pallas import tpu as pltpu
from jax.experimental.pallas import tpu_sc as plsc

N_PAIR = 8
N_ENTI = 36
N_CTX = 10
MAX_L = 40
SUF_L = MAX_L - 1 - N_CTX
N_CLS = 132
D = 768
LD = MAX_L * D
CTX_W = N_CTX * D
SUF_W = SUF_L * D
CB = 33
N_CT = N_CLS // CB

NUM_WORKERS = 32
MAX_CPW = 5


def _ctx_body(ids_ref, enti_ref, w1_ref, b1_ref, w2_ref, meta_ref,
              subj_ref, obj_ref, ctx_s_ref, ctx_o_ref):
    ids = ids_ref[...]
    iota = lax.broadcasted_iota(jnp.int32, (N_PAIR, N_ENTI), 1)
    s_oh = (ids[:, 0:1] == iota).astype(jnp.float32)
    o_oh = (ids[:, 1:2] == iota).astype(jnp.float32)
    enti = enti_ref[...]
    s_embd = jnp.dot(s_oh, enti, preferred_element_type=jnp.float32)
    o_embd = jnp.dot(o_oh, enti, preferred_element_type=jnp.float32)
    so = jnp.concatenate([s_embd, o_embd], axis=-1)
    h = jax.nn.relu(jnp.dot(so, w1_ref[...],
                            preferred_element_type=jnp.float32) + b1_ref[...])
    q = jnp.dot(h, w2_ref[...], preferred_element_type=jnp.float32)
    meta = meta_ref[...]
    scale = 1.0 / math.sqrt(D)

    def attn(qq):
        logits = lax.dot_general(qq, meta, (((1,), (1,)), ((), ()))) * scale
        probs = jax.nn.softmax(logits, axis=-1)
        return jnp.dot(probs, meta, preferred_element_type=jnp.float32)

    s_ctx = attn(q[:, :D])
    o_ctx = attn(q[:, D:])
    ctx_s_ref[...] = subj_ref[...] + jnp.tile(s_ctx, (1, N_CTX))
    ctx_o_ref[...] = obj_ref[...] + jnp.tile(o_ctx, (1, N_CTX))


_CTX_CALL = pl.pallas_call(
    _ctx_body,
    out_shape=[jax.ShapeDtypeStruct((N_PAIR, CTX_W), jnp.float32),
               jax.ShapeDtypeStruct((N_PAIR, CTX_W), jnp.float32)],
)


def _asm_body(prefix_ref, suffix_ref, ctx_ref, out_ref):
    out_ref[0, :, 0:D] = prefix_ref[0]
    out_ref[0, :, D:D + CTX_W] = jnp.broadcast_to(ctx_ref[0], (CB, CTX_W))
    out_ref[0, :, D + CTX_W:LD] = suffix_ref[0]


_ASM_CALL = pl.pallas_call(
    _asm_body,
    grid=(N_CT, N_PAIR),
    in_specs=[
        pl.BlockSpec((1, CB, D), lambda ct, p: (ct, 0, 0)),
        pl.BlockSpec((1, CB, SUF_W), lambda ct, p: (ct, 0, 0)),
        pl.BlockSpec((1, 1, CTX_W), lambda ct, p: (p, 0, 0)),
    ],
    out_specs=pl.BlockSpec((1, CB, LD), lambda ct, p: (p * N_CT + ct, 0, 0)),
    out_shape=jax.ShapeDtypeStruct((N_PAIR * N_CT, CB, LD), jnp.float32),
)


def _sc_obj_body(prefix_hbm, suffix_hbm, ctx_hbm, out_hbm,
                 ctx_buf, pfx_buf, sfx_buf, sem_in, sem_pw, sem_sw, sem_cw):
    wid = lax.axis_index("s") * 2 + lax.axis_index("c")
    nc = jnp.where(wid < 4, 5, 4)
    c_base = wid * 4 + jnp.minimum(wid, 4)

    pltpu.sync_copy(ctx_hbm, ctx_buf)

    def start_load(k, slot):
        c = c_base + k
        pltpu.make_async_copy(prefix_hbm.at[c], pfx_buf.at[slot],
                              sem_in.at[slot]).start()
        pltpu.make_async_copy(suffix_hbm.at[c], sfx_buf.at[slot],
                              sem_in.at[slot]).start()

    def wait_load(slot):
        pltpu.make_async_copy(prefix_hbm.at[0], pfx_buf.at[slot],
                              sem_in.at[slot]).wait()
        pltpu.make_async_copy(suffix_hbm.at[0], sfx_buf.at[slot],
                              sem_in.at[slot]).wait()

    def issue_writes(k, slot):
        c = c_base + k
        for p in range(N_PAIR):
            r = p * N_CLS + c
            pltpu.make_async_copy(pfx_buf.at[slot],
                                  out_hbm.at[r, pl.ds(0, D)],
                                  sem_pw.at[slot]).start()
            pltpu.make_async_copy(ctx_buf.at[p],
                                  out_hbm.at[r, pl.ds(D, CTX_W)],
                                  sem_cw).start()
            pltpu.make_async_copy(sfx_buf.at[slot],
                                  out_hbm.at[r, pl.ds(D + CTX_W, SUF_W)],
                                  sem_sw.at[slot]).start()

    def wait_writes(slot):
        for _ in range(N_PAIR):
            pltpu.make_async_copy(pfx_buf.at[slot],
                                  out_hbm.at[0, pl.ds(0, D)],
                                  sem_pw.at[slot]).wait()
            pltpu.make_async_copy(sfx_buf.at[slot],
                                  out_hbm.at[0, pl.ds(D + CTX_W, SUF_W)],
                                  sem_sw.at[slot]).wait()

    start_load(0, 0)
    for k in range(MAX_CPW):
        slot = k % 2
        nslot = (k + 1) % 2

        @pl.when(k < nc)
        def _body():
            if k + 1 < MAX_CPW:
                @pl.when(k + 1 < nc)
                def _prefetch():
                    if k >= 1:
                        wait_writes(nslot)
                    start_load(k + 1, nslot)
            wait_load(slot)
            issue_writes(k, slot)

    for k in range(MAX_CPW):
        @pl.when((k < nc) & (k >= nc - 2))
        def _drain():
            wait_writes(k % 2)

    for k in range(MAX_CPW):
        @pl.when(k < nc)
        def _drain_ctx():
            for _ in range(N_PAIR):
                pltpu.make_async_copy(ctx_buf.at[0],
                                      out_hbm.at[0, pl.ds(D, CTX_W)],
                                      sem_cw).wait()


_SC_OBJ = functools.partial(
    pl.kernel,
    out_type=jax.ShapeDtypeStruct((N_PAIR * N_CLS, LD), jnp.float32),
    mesh=plsc.VectorSubcoreMesh(core_axis_name="c", subcore_axis_name="s"),
    scratch_types=[
        pltpu.VMEM((N_PAIR, CTX_W), jnp.float32),
        pltpu.VMEM((2, D), jnp.float32),
        pltpu.VMEM((2, SUF_W), jnp.float32),
        pltpu.SemaphoreType.DMA((2,)),
        pltpu.SemaphoreType.DMA((2,)),
        pltpu.SemaphoreType.DMA((2,)),
        pltpu.SemaphoreType.DMA,
    ],
)(_sc_obj_body)


@jax.jit
def kernel(so_cls_ids, enti_txt_embds, W1, b1, W2, meta_ctx_embds,
           subj_ctx_embds, obj_ctx_embds, prefix_embds, suffix_embds,
           token_mask):
    prefix = prefix_embds[1:1 + N_CLS].reshape(N_CLS, D)
    suffix = suffix_embds[1:1 + N_CLS].reshape(N_CLS, SUF_W)
    ctx_s, ctx_o = _CTX_CALL(so_cls_ids, enti_txt_embds, W1,
                             b1.reshape(1, 256), W2, meta_ctx_embds,
                             subj_ctx_embds.reshape(1, CTX_W),
                             obj_ctx_embds.reshape(1, CTX_W))
    out_s = _ASM_CALL(prefix.reshape(N_CT, CB, D),
                      suffix.reshape(N_CT, CB, SUF_W),
                      ctx_s.reshape(N_PAIR, 1, CTX_W))
    out_o = _SC_OBJ(prefix, suffix, ctx_o)
    out_s = out_s.reshape(N_PAIR * N_CLS, MAX_L, D)
    out_o = out_o.reshape(N_PAIR * N_CLS, MAX_L, D)
    tm_rep = jnp.tile(token_mask[1:1 + N_CLS], (N_PAIR, 1))
    return out_s, out_o, tm_rep


# speedup vs baseline: 4.1305x; 4.1305x over previous
import math

import jax
import jax.numpy as jnp
from jax import lax
from jax.experimental import pallas as pl

N_PAIR = 8
N_ENTI = 36
N_CTX = 10
MAX_L = 40
SUF_L = MAX_L - 1 - N_CTX
N_CLS = 132
D = 768
CB = 33
N_CT = N_CLS // CB


def _ctx_body(ids_ref, enti_ref, w1_ref, b1_ref, w2_ref, meta_ref,
              subj_ref, obj_ref, ctx_s_ref, ctx_o_ref):
    ids = ids_ref[...]
    iota = lax.broadcasted_iota(jnp.int32, (N_PAIR, N_ENTI), 1)
    s_oh = (ids[:, 0:1] == iota).astype(jnp.float32)
    o_oh = (ids[:, 1:2] == iota).astype(jnp.float32)
    enti = enti_ref[...]
    s_embd = jnp.dot(s_oh, enti, preferred_element_type=jnp.float32)
    o_embd = jnp.dot(o_oh, enti, preferred_element_type=jnp.float32)
    so = jnp.concatenate([s_embd, o_embd], axis=-1)
    h = jax.nn.relu(jnp.dot(so, w1_ref[...],
                            preferred_element_type=jnp.float32) + b1_ref[...])
    q = jnp.dot(h, w2_ref[...], preferred_element_type=jnp.float32)
    meta = meta_ref[...]
    scale = 1.0 / math.sqrt(D)

    def attn(qq):
        logits = lax.dot_general(qq, meta, (((1,), (1,)), ((), ()))) * scale
        probs = jax.nn.softmax(logits, axis=-1)
        return jnp.dot(probs, meta, preferred_element_type=jnp.float32)

    s_ctx = attn(q[:, :D])
    o_ctx = attn(q[:, D:])
    ctx_s_ref[...] = subj_ref[...][None, :, :] + s_ctx[:, None, :]
    ctx_o_ref[...] = obj_ref[...][None, :, :] + o_ctx[:, None, :]


_CTX_CALL = pl.pallas_call(
    _ctx_body,
    out_shape=[jax.ShapeDtypeStruct((N_PAIR, N_CTX, D), jnp.float32),
               jax.ShapeDtypeStruct((N_PAIR, N_CTX, D), jnp.float32)],
)


def _asm_body(prefix_ref, suffix_ref, ctx_s_ref, ctx_o_ref,
              out_s_ref, out_o_ref):
    pr = prefix_ref[...]
    sf = suffix_ref[...]
    cs = jnp.broadcast_to(ctx_s_ref[...], (CB, N_CTX, D))
    co = jnp.broadcast_to(ctx_o_ref[...], (CB, N_CTX, D))
    out_s_ref[...] = jnp.concatenate([pr, cs, sf], axis=1)
    out_o_ref[...] = jnp.concatenate([pr, co, sf], axis=1)


_ASM_CALL = pl.pallas_call(
    _asm_body,
    grid=(N_CT, N_PAIR),
    in_specs=[
        pl.BlockSpec((CB, 1, D), lambda ct, p: (ct, 0, 0)),
        pl.BlockSpec((CB, SUF_L, D), lambda ct, p: (ct, 0, 0)),
        pl.BlockSpec((1, N_CTX, D), lambda ct, p: (p, 0, 0)),
        pl.BlockSpec((1, N_CTX, D), lambda ct, p: (p, 0, 0)),
    ],
    out_specs=[
        pl.BlockSpec((CB, MAX_L, D), lambda ct, p: (p * N_CT + ct, 0, 0)),
        pl.BlockSpec((CB, MAX_L, D), lambda ct, p: (p * N_CT + ct, 0, 0)),
    ],
    out_shape=[
        jax.ShapeDtypeStruct((N_PAIR * N_CLS, MAX_L, D), jnp.float32),
        jax.ShapeDtypeStruct((N_PAIR * N_CLS, MAX_L, D), jnp.float32)],
)


@jax.jit
def kernel(so_cls_ids, enti_txt_embds, W1, b1, W2, meta_ctx_embds,
           subj_ctx_embds, obj_ctx_embds, prefix_embds, suffix_embds,
           token_mask):
    prefix = prefix_embds[1:1 + N_CLS]
    suffix = suffix_embds[1:1 + N_CLS]
    ctx_s, ctx_o = _CTX_CALL(so_cls_ids, enti_txt_embds, W1,
                             b1.reshape(1, 256), W2, meta_ctx_embds,
                             subj_ctx_embds, obj_ctx_embds)
    out_s, out_o = _ASM_CALL(prefix, suffix, ctx_s, ctx_o)
    tm_rep = jnp.tile(token_mask[1:1 + N_CLS], (N_PAIR, 1))
    return out_s, out_o, tm_rep


# speedup vs baseline: 4.6057x; 1.1150x over previous
import math

import jax
import jax.numpy as jnp
from jax import lax
from jax.experimental import pallas as pl

N_PAIR = 8
N_ENTI = 36
N_CTX = 10
MAX_L = 40
SUF_L = MAX_L - 1 - N_CTX
N_CLS = 132
D = 768
CB = 33
N_CT = N_CLS // CB


def _ctx_body(ids_ref, enti_ref, w1_ref, b1_ref, w2_ref, meta_ref,
              subj_ref, obj_ref, ctx_s_ref, ctx_o_ref):
    ids = ids_ref[...]
    iota = lax.broadcasted_iota(jnp.int32, (N_PAIR, N_ENTI), 1)
    s_oh = (ids[:, 0:1] == iota).astype(jnp.float32)
    o_oh = (ids[:, 1:2] == iota).astype(jnp.float32)
    enti = enti_ref[...]
    s_embd = jnp.dot(s_oh, enti, preferred_element_type=jnp.float32)
    o_embd = jnp.dot(o_oh, enti, preferred_element_type=jnp.float32)
    so = jnp.concatenate([s_embd, o_embd], axis=-1)
    h = jax.nn.relu(jnp.dot(so, w1_ref[...],
                            preferred_element_type=jnp.float32) + b1_ref[...])
    q = jnp.dot(h, w2_ref[...], preferred_element_type=jnp.float32)
    meta = meta_ref[...]
    scale = 1.0 / math.sqrt(D)

    def attn(qq):
        logits = lax.dot_general(qq, meta, (((1,), (1,)), ((), ()))) * scale
        probs = jax.nn.softmax(logits, axis=-1)
        return jnp.dot(probs, meta, preferred_element_type=jnp.float32)

    s_ctx = attn(q[:, :D])
    o_ctx = attn(q[:, D:])
    ctx_s_ref[...] = subj_ref[...][None, :, :] + s_ctx[:, None, :]
    ctx_o_ref[...] = obj_ref[...][None, :, :] + o_ctx[:, None, :]


_CTX_CALL = pl.pallas_call(
    _ctx_body,
    out_shape=[jax.ShapeDtypeStruct((N_PAIR, N_CTX, D), jnp.float32),
               jax.ShapeDtypeStruct((N_PAIR, N_CTX, D), jnp.float32)],
)


def _asm_body(prefix_ref, suffix_ref, ctx_s_ref, ctx_o_ref,
              out_s_ref, out_o_ref):
    ct = pl.program_id(0)
    pr = prefix_ref[pl.ds(1 + ct * CB, CB)]
    sf = suffix_ref[pl.ds(1 + ct * CB, CB)]
    cs = jnp.broadcast_to(ctx_s_ref[...], (CB, N_CTX, D))
    co = jnp.broadcast_to(ctx_o_ref[...], (CB, N_CTX, D))
    out_s_ref[...] = jnp.concatenate([pr, cs, sf], axis=1)
    out_o_ref[...] = jnp.concatenate([pr, co, sf], axis=1)


_ASM_CALL = pl.pallas_call(
    _asm_body,
    grid=(N_CT, N_PAIR),
    in_specs=[
        pl.BlockSpec((1 + N_CLS, 1, D), lambda ct, p: (0, 0, 0)),
        pl.BlockSpec((1 + N_CLS, SUF_L, D), lambda ct, p: (0, 0, 0)),
        pl.BlockSpec((1, N_CTX, D), lambda ct, p: (p, 0, 0)),
        pl.BlockSpec((1, N_CTX, D), lambda ct, p: (p, 0, 0)),
    ],
    out_specs=[
        pl.BlockSpec((CB, MAX_L, D), lambda ct, p: (p * N_CT + ct, 0, 0)),
        pl.BlockSpec((CB, MAX_L, D), lambda ct, p: (p * N_CT + ct, 0, 0)),
    ],
    out_shape=[
        jax.ShapeDtypeStruct((N_PAIR * N_CLS, MAX_L, D), jnp.float32),
        jax.ShapeDtypeStruct((N_PAIR * N_CLS, MAX_L, D), jnp.float32)],
)


@jax.jit
def kernel(so_cls_ids, enti_txt_embds, W1, b1, W2, meta_ctx_embds,
           subj_ctx_embds, obj_ctx_embds, prefix_embds, suffix_embds,
           token_mask):
    ctx_s, ctx_o = _CTX_CALL(so_cls_ids, enti_txt_embds, W1,
                             b1.reshape(1, 256), W2, meta_ctx_embds,
                             subj_ctx_embds, obj_ctx_embds)
    out_s, out_o = _ASM_CALL(prefix_embds, suffix_embds, ctx_s, ctx_o)
    tm_rep = jnp.tile(token_mask[1:1 + N_CLS], (N_PAIR, 1))
    return out_s, out_o, tm_rep


# speedup vs baseline: 4.7146x; 1.0237x over previous
import math

import jax
import jax.numpy as jnp
from jax import lax
from jax.experimental import pallas as pl
from jax.experimental.pallas import tpu as pltpu

N_PAIR = 8
N_ENTI = 36
N_CTX = 10
MAX_L = 40
SUF_L = MAX_L - 1 - N_CTX
N_CLS = 132
D = 768
CB = 33
N_CT = N_CLS // CB


def _fused_body(ids_ref, enti_ref, w1_ref, b1_ref, w2_ref, meta_ref,
                subj_ref, obj_ref, prefix_ref, suffix_ref,
                out_s_ref, out_o_ref, ctx_s_buf, ctx_o_buf):
    ct = pl.program_id(0)
    p = pl.program_id(1)

    @pl.when((ct == 0) & (p == 0))
    def _compute_ctx():
        ids = ids_ref[...]
        iota = lax.broadcasted_iota(jnp.int32, (N_PAIR, N_ENTI), 1)
        s_oh = (ids[:, 0:1] == iota).astype(jnp.float32)
        o_oh = (ids[:, 1:2] == iota).astype(jnp.float32)
        enti = enti_ref[...]
        s_embd = jnp.dot(s_oh, enti, preferred_element_type=jnp.float32)
        o_embd = jnp.dot(o_oh, enti, preferred_element_type=jnp.float32)
        so = jnp.concatenate([s_embd, o_embd], axis=-1)
        h = jax.nn.relu(jnp.dot(so, w1_ref[...],
                                preferred_element_type=jnp.float32)
                        + b1_ref[...])
        q = jnp.dot(h, w2_ref[...], preferred_element_type=jnp.float32)
        meta = meta_ref[...]
        scale = 1.0 / math.sqrt(D)

        def attn(qq):
            logits = lax.dot_general(qq, meta,
                                     (((1,), (1,)), ((), ()))) * scale
            probs = jax.nn.softmax(logits, axis=-1)
            return jnp.dot(probs, meta, preferred_element_type=jnp.float32)

        s_ctx = attn(q[:, :D])
        o_ctx = attn(q[:, D:])
        ctx_s_buf[...] = subj_ref[...][None, :, :] + s_ctx[:, None, :]
        ctx_o_buf[...] = obj_ref[...][None, :, :] + o_ctx[:, None, :]

    pr = prefix_ref[pl.ds(1 + ct * CB, CB)]
    sf = suffix_ref[pl.ds(1 + ct * CB, CB)]
    cs = jnp.broadcast_to(ctx_s_buf[pl.ds(p, 1)], (CB, N_CTX, D))
    co = jnp.broadcast_to(ctx_o_buf[pl.ds(p, 1)], (CB, N_CTX, D))
    out_s_ref[...] = jnp.concatenate([pr, cs, sf], axis=1)
    out_o_ref[...] = jnp.concatenate([pr, co, sf], axis=1)


_FUSED = pl.pallas_call(
    _fused_body,
    grid=(N_CT, N_PAIR),
    in_specs=[
        pl.BlockSpec((N_PAIR, 2), lambda ct, p: (0, 0)),
        pl.BlockSpec((N_ENTI, 256), lambda ct, p: (0, 0)),
        pl.BlockSpec((512, 256), lambda ct, p: (0, 0)),
        pl.BlockSpec((1, 256), lambda ct, p: (0, 0)),
        pl.BlockSpec((256, 2 * D), lambda ct, p: (0, 0)),
        pl.BlockSpec((N_CTX, D), lambda ct, p: (0, 0)),
        pl.BlockSpec((N_CTX, D), lambda ct, p: (0, 0)),
        pl.BlockSpec((N_CTX, D), lambda ct, p: (0, 0)),
        pl.BlockSpec((1 + N_CLS, 1, D), lambda ct, p: (0, 0, 0)),
        pl.BlockSpec((1 + N_CLS, SUF_L, D), lambda ct, p: (0, 0, 0)),
    ],
    out_specs=[
        pl.BlockSpec((CB, MAX_L, D), lambda ct, p: (p * N_CT + ct, 0, 0)),
        pl.BlockSpec((CB, MAX_L, D), lambda ct, p: (p * N_CT + ct, 0, 0)),
    ],
    out_shape=[
        jax.ShapeDtypeStruct((N_PAIR * N_CLS, MAX_L, D), jnp.float32),
        jax.ShapeDtypeStruct((N_PAIR * N_CLS, MAX_L, D), jnp.float32)],
    scratch_shapes=[
        pltpu.VMEM((N_PAIR, N_CTX, D), jnp.float32),
        pltpu.VMEM((N_PAIR, N_CTX, D), jnp.float32),
    ],
)


@jax.jit
def kernel(so_cls_ids, enti_txt_embds, W1, b1, W2, meta_ctx_embds,
           subj_ctx_embds, obj_ctx_embds, prefix_embds, suffix_embds,
           token_mask):
    out_s, out_o = _FUSED(so_cls_ids, enti_txt_embds, W1, b1.reshape(1, 256),
                          W2, meta_ctx_embds, subj_ctx_embds, obj_ctx_embds,
                          prefix_embds, suffix_embds)
    tm_rep = jnp.tile(token_mask[1:1 + N_CLS], (N_PAIR, 1))
    return out_s, out_o, tm_rep
